# baseline (speedup 1.0000x reference)
import functools

import jax
import jax.numpy as jnp
from jax import lax
from jax.experimental import pallas as pl
from jax.experimental.pallas import tpu as pltpu
from jax.experimental.pallas import tpu_sc as plsc

B = 16384
BAG = 50
D = 64
HD = D // 2
V = 1000000

Q = 262144
VT = 4 * Q

NC = 2
NS = 16
NW = NC * NS

BAGS_PER_W = B // NW
ROWS_PER_W = BAGS_PER_W * BAG
CHUNK_BAGS = 8
CHUNK_ROWS = CHUNK_BAGS * BAG
N_CHUNKS = BAGS_PER_W // CHUNK_BAGS
SUB = 80
N_SUB = CHUNK_ROWS // SUB

VB3 = 8192


def _tp_body(in0, in1, in2, in3, out_ref):
    stacked = jnp.concatenate(
        [in0[...], in1[...], in2[...], in3[...]], axis=0)
    out_ref[...] = stacked.T


def _tc_transpose_half(wt, h):
    nb = Q // VB3
    last = (V - 1) // VB3

    def imap(q):
        if q < 3:
            return lambda i: (h, q * nb + i)
        return lambda i: (h, jnp.minimum(3 * nb + i, last))

    return pl.pallas_call(
        _tp_body,
        grid=(nb,),
        in_specs=[pl.BlockSpec((HD, VB3), imap(q)) for q in range(4)],
        out_specs=pl.BlockSpec((VB3, 4 * HD), lambda i: (i, 0)),
        out_shape=jax.ShapeDtypeStruct((Q, 4 * HD), jnp.float32),
    )(wt, wt, wt, wt)


def _fire(w_hbm, idx_v, buf, sem, g):
    base = g * CHUNK_ROWS
    for s in range(N_SUB):
        pltpu.async_copy(
            w_hbm.at[idx_v.at[pl.ds(base + s * SUB, SUB)]],
            buf.at[pl.ds(s * SUB, SUB)],
            sem,
        )


def _drain(w_hbm, buf, sem):
    pltpu.make_async_copy(w_hbm.at[pl.ds(0, CHUNK_ROWS)], buf, sem).wait()


def _accumulate(buf, ob, bias_vecs):
    def bag_body(bb, carry):
        row0 = bb * BAG

        def body(j, accs):
            r = row0 + j
            return tuple(
                accs[k] + buf[r, pl.ds(16 * k, 16)] for k in range(2)
            )

        accs = lax.fori_loop(0, BAG, body, bias_vecs, unroll=10)
        for k in range(2):
            ob[bb, pl.ds(16 * k, 16)] = accs[k]
        return carry

    lax.fori_loop(0, CHUNK_BAGS, bag_body, 0)


def _remap_chunk(idx_v, g):
    n = CHUNK_ROWS // 16
    base = g * n

    def body(i, carry):
        off = 16 * (base + i)
        v = idx_v[pl.ds(off, 16)]
        j = lax.shift_left(jnp.bitwise_and(v, Q - 1), 2) + lax.shift_right_logical(v, 18)
        idx_v[pl.ds(off, 16)] = j
        return carry

    lax.fori_loop(0, n, body, 0, unroll=5)


def _sc_body(h, idx_hbm, w_hbm, bias_hbm, out_hbm,
             idx_v, rows0, rows1, rows2, rows3,
             ob0, ob1, ob2, ob3, bias_v,
             sem0, sem1, sem2, sem3, osem):
    wid = lax.axis_index("s") * NC + lax.axis_index("c")
    bag0 = wid * BAGS_PER_W

    pltpu.sync_copy(bias_hbm, bias_v)
    pltpu.sync_copy(idx_hbm.at[pl.ds(wid * ROWS_PER_W, ROWS_PER_W)], idx_v)

    bias_vecs = tuple(bias_v[pl.ds(32 * h + 16 * k, 16)] for k in range(2))
    bufs = (rows0, rows1, rows2, rows3)
    sems = (sem0, sem1, sem2, sem3)
    obs = (ob0, ob1, ob2, ob3)

    def wait_one_out():
        pltpu.make_async_copy(
            out_hbm.at[pl.ds(0, CHUNK_BAGS)], obs[0], osem).wait()

    def step(g, b, fire_next, out_wait):
        if out_wait:
            wait_one_out()
        if fire_next:
            _remap_chunk(idx_v, g + 3)
            _fire(w_hbm, idx_v, bufs[(b + 3) % 4], sems[(b + 3) % 4], g + 3)
        _drain(w_hbm, bufs[b], sems[b])
        _accumulate(bufs[b], obs[b], bias_vecs)
        pltpu.async_copy(
            obs[b],
            out_hbm.at[pl.ds(bag0 + g * CHUNK_BAGS, CHUNK_BAGS)],
            osem,
        )

    for g in range(3):
        _remap_chunk(idx_v, g)
        _fire(w_hbm, idx_v, bufs[g], sems[g], g)

    for b in range(4):
        step(b, b, fire_next=True, out_wait=False)

    def chunk_quad(i, carry):
        for b in range(4):
            g = 4 * (i + 1) + b
            step(g, b, fire_next=True, out_wait=True)
        return carry

    lax.fori_loop(0, (N_CHUNKS - 8) // 4, chunk_quad, 0)

    step(N_CHUNKS - 4, 0, fire_next=True, out_wait=True)
    for b in range(1, 4):
        step(N_CHUNKS - 4 + b, b, fire_next=False, out_wait=True)

    for _ in range(4):
        wait_one_out()


def _sc_gather_half(indices, w_rows, bias, h):
    mesh = plsc.VectorSubcoreMesh(core_axis_name="c", subcore_axis_name="s")
    run = pl.kernel(
        functools.partial(_sc_body, h),
        out_type=jax.ShapeDtypeStruct((B, HD), jnp.float32),
        mesh=mesh,
        scratch_types=[
            pltpu.VMEM((ROWS_PER_W,), jnp.int32),
            pltpu.VMEM((CHUNK_ROWS, HD), jnp.float32),
            pltpu.VMEM((CHUNK_ROWS, HD), jnp.float32),
            pltpu.VMEM((CHUNK_ROWS, HD), jnp.float32),
            pltpu.VMEM((CHUNK_ROWS, HD), jnp.float32),
            pltpu.VMEM((CHUNK_BAGS, HD), jnp.float32),
            pltpu.VMEM((CHUNK_BAGS, HD), jnp.float32),
            pltpu.VMEM((CHUNK_BAGS, HD), jnp.float32),
            pltpu.VMEM((CHUNK_BAGS, HD), jnp.float32),
            pltpu.VMEM((D,), jnp.float32),
            pltpu.SemaphoreType.DMA,
            pltpu.SemaphoreType.DMA,
            pltpu.SemaphoreType.DMA,
            pltpu.SemaphoreType.DMA,
            pltpu.SemaphoreType.DMA,
        ],
        compiler_params=pltpu.CompilerParams(use_tc_tiling_on_sc=False),
    )
    return run(indices, w_rows, bias)


@jax.jit
def _embed_bag(indices, w, bias):
    wt = w.T
    t0 = _tc_transpose_half(wt, 0)
    o0 = _sc_gather_half(indices, t0.reshape(VT, HD), bias, 0)
    t1 = _tc_transpose_half(wt, 1)
    o1 = _sc_gather_half(indices, t1.reshape(VT, HD), bias, 1)
    return jnp.concatenate([o0, o1], axis=1)


def kernel(indices, offsets, W, bias):
    del offsets
    return _embed_bag(indices.astype(jnp.int32), W, bias)
